# baseline (speedup 1.0000x reference)
import functools

import jax
import jax.numpy as jnp
from jax import lax
from jax.experimental import pallas as pl
from jax.experimental.pallas import tpu as pltpu
from jax.experimental.pallas import tpu_sc as plsc

_N = 10000
_E = 320000
_D = 128
_NC = 2
_NS = 16
_NW = _NC * _NS
_EPW = _E // _NW
_C = 128
_G = _EPW // _C
_CT = _EPW - _G * _C
_RPT = 624
_TAIL = _N - _RPT * _NS
_ZB = 48

assert _EPW * _NW == _E and 0 <= _TAIL < _RPT
assert _G % 2 == 0 and _G >= 4 and 0 < _CT and _CT % 8 == 0
assert _RPT % 8 == 0 and _TAIL % 8 == 0
assert _RPT % _ZB == 0 and _TAIL <= _ZB
_ZG = _RPT // _ZB
_RCNT = _N // 16
assert _N % 16 == 0 and _CT == 16


def _make_agg():
    out_type = [jax.ShapeDtypeStruct((_NC, _N, _D), jnp.float32)]
    scratch = [
        pltpu.VMEM((2, _C), jnp.int32),
        pltpu.VMEM((2, _C), jnp.int32),
        pltpu.VMEM((_C, _D), jnp.float32),
        pltpu.VMEM((_C, _D), jnp.float32),
        pltpu.VMEM((_CT,), jnp.int32),
        pltpu.VMEM((_CT,), jnp.int32),
        pltpu.VMEM((_CT, _D), jnp.float32),
        pltpu.VMEM((_ZB, _D), jnp.float32),
        pltpu.VMEM_SHARED((_N, _D), jnp.float32),
        pltpu.SemaphoreType.DMA,
        pltpu.SemaphoreType.DMA,
        pltpu.SemaphoreType.DMA,
        pltpu.SemaphoreType.DMA,
    ]

    mesh = plsc.VectorSubcoreMesh(core_axis_name="c", subcore_axis_name="s")

    def body(*refs):
        (feats, srch, dsth, zrow,
         aggout,
         srcv, dstv, rows0, rows1, srct, dstt, rowst,
         bounce, aggsh, sem0, sem1, ssem0, ssem1) = refs
        rows = (rows0, rows1)
        sems = (sem0, sem1)
        ssems = (ssem0, ssem1)
        cid = lax.axis_index("c")
        sid = lax.axis_index("s")
        wid = cid * _NS + sid
        r0 = sid * _RPT
        ebase = wid * _EPW

        def zero_slice():
            pltpu.sync_copy(zrow, bounce)
            for j in range(_ZG):
                pltpu.sync_copy(bounce, aggsh.at[pl.ds(r0 + j * _ZB, _ZB)])

            @pl.when(sid == _NS - 1)
            def _():
                pltpu.sync_copy(bounce.at[pl.ds(0, _TAIL)],
                                aggsh.at[pl.ds(_RPT * _NS, _TAIL)])

        def export_slice(out):
            for j in range(_ZG):
                pltpu.sync_copy(aggsh.at[pl.ds(r0 + j * _ZB, _ZB)], bounce)
                pltpu.sync_copy(bounce, out.at[cid, pl.ds(r0 + j * _ZB, _ZB)])

            @pl.when(sid == _NS - 1)
            def _():
                pltpu.sync_copy(aggsh.at[pl.ds(_RPT * _NS, _TAIL)],
                                bounce.at[pl.ds(0, _TAIL)])
                pltpu.sync_copy(bounce.at[pl.ds(0, _TAIL)],
                                out.at[cid, pl.ds(_RPT * _NS, _TAIL)])

        def load_idx(off, slot):
            pltpu.sync_copy(srch.at[pl.ds(off, _C)], srcv.at[slot])
            pltpu.sync_copy(dsth.at[pl.ds(off, _C)], dstv.at[slot])

        def gather(slot):
            return pltpu.make_async_copy(feats.at[srcv.at[slot]],
                                         rows[slot], sems[slot])

        def scat(slot):
            return pltpu.make_async_copy(rows[slot], aggsh.at[dstv.at[slot]],
                                         ssems[slot])

        zero_slice()
        plsc.subcore_barrier()

        load_idx(ebase, 0)
        gather(0).start()
        load_idx(ebase + _C, 1)
        gather(1).start()

        def step(g, carry):
            ca = 2 * g
            gather(0).wait()
            scat(0).start(add=True)
            gather(1).wait()
            scat(1).start(add=True)
            scat(0).wait()
            load_idx(ebase + (ca + 2) * _C, 0)
            gather(0).start()
            scat(1).wait()
            load_idx(ebase + (ca + 3) * _C, 1)
            gather(1).start()
            return carry

        lax.fori_loop(0, (_G - 2) // 2, step, 0)
        gather(0).wait()
        scat(0).start(add=True)
        gather(1).wait()
        scat(1).start(add=True)
        tbase = ebase + _G * _C
        pltpu.sync_copy(srch.at[pl.ds(tbase, _CT)], srct)
        pltpu.sync_copy(dsth.at[pl.ds(tbase, _CT)], dstt)
        pltpu.make_async_copy(feats.at[srct], rowst, sem0).start()
        scat(0).wait()
        scat(1).wait()
        pltpu.make_async_copy(feats.at[srct], rowst, sem0).wait()
        pltpu.sync_copy(rowst, aggsh.at[dstt], add=True)
        plsc.subcore_barrier()
        export_slice(aggout)

    return pl.kernel(body, mesh=mesh, out_type=out_type,
                     scratch_types=scratch)


def _make_cnt():
    CC = 2000

    def body(dsth, cntout, dstv, cnt2d, sem):
        cid = lax.axis_index("c")
        sid = lax.axis_index("s")
        wid = cid * _NS + sid
        ebase = wid * _EPW
        z16 = jnp.zeros((16,), jnp.float32)
        ones16 = jnp.ones((16,), jnp.float32)

        def zc(i, carry):
            cnt2d[i, pl.ds(0, 16)] = z16
            return carry

        lax.fori_loop(0, _RCNT, zc, 0)

        def kstep(k, carry):
            pltpu.sync_copy(dsth.at[pl.ds(ebase + k * CC, CC)], dstv)
            for i in range(CC // 16):
                dvec = dstv[pl.ds(i * 16, 16)]
                ri = lax.shift_right_logical(dvec, 4)
                li = lax.bitwise_and(dvec, 15)
                plsc.addupdate_scatter(cnt2d, [ri, li], ones16)
            return carry

        lax.fori_loop(0, _EPW // CC, kstep, 0)
        pltpu.sync_copy(cnt2d, cntout.at[wid])

    mesh = plsc.VectorSubcoreMesh(core_axis_name="c", subcore_axis_name="s")
    return pl.kernel(
        body, mesh=mesh,
        out_type=[jax.ShapeDtypeStruct((_NW, _RCNT, 16), jnp.float32)],
        compiler_params=pltpu.CompilerParams(needs_layout_passes=False),
        scratch_types=[
            pltpu.VMEM((CC,), jnp.int32),
            pltpu.VMEM((_RCNT, 16), jnp.float32),
            pltpu.SemaphoreType.DMA,
        ])


_agg_plain = _make_agg()
_cnt_parts = _make_cnt()

_BN = 1000


def _merged_counts(c_ref, i):
    s = jnp.sum(c_ref[...], axis=0)
    n_row = jax.lax.broadcasted_iota(jnp.int32, (_BN, _RCNT), 0) + i * _BN
    r_col = jax.lax.broadcasted_iota(jnp.int32, (_BN, _RCNT), 1)
    pick_row = (lax.shift_right_logical(n_row, 4) == r_col)
    t = lax.dot_general(pick_row.astype(jnp.float32), s,
                        (((1,), (0,)), ((), ())),
                        preferred_element_type=jnp.float32)
    n_lane = jax.lax.broadcasted_iota(jnp.int32, (_BN, 16), 0) + i * _BN
    l_lane = jax.lax.broadcasted_iota(jnp.int32, (_BN, 16), 1)
    sel = jnp.where(lax.bitwise_and(n_lane, 15) == l_lane, t, 0.0)
    return jnp.sum(sel, axis=1, keepdims=True)


def _sage_out(agg_ref, cnt, x_ref, wl_ref, b_ref, wr_ref, relu):
    agg = agg_ref[0] + agg_ref[1]
    mean = agg / jnp.maximum(cnt, 1.0)
    dn = (((1,), (0,)), ((), ()))
    acc = (lax.dot_general(mean, wl_ref[...], dn,
                           preferred_element_type=jnp.float32,
                           precision=lax.Precision.HIGHEST)
           + b_ref[...]
           + lax.dot_general(x_ref[...], wr_ref[...], dn,
                             preferred_element_type=jnp.float32,
                             precision=lax.Precision.HIGHEST))
    return jnp.maximum(acc, 0.0) if relu else acc


def _dense1_body(agg_ref, c_ref, x_ref, wl_ref, b_ref, wr_ref,
                 o_ref, cnt8_ref):
    i = pl.program_id(0)
    cnt = _merged_counts(c_ref, i)
    cnt8_ref[...] = jnp.broadcast_to(cnt, (_BN, 8))
    o_ref[...] = _sage_out(agg_ref, cnt, x_ref, wl_ref, b_ref, wr_ref, True)


def _dense1(aggp, cparts, feats, wl, b, wr):
    return pl.pallas_call(
        _dense1_body,
        grid=(_N // _BN,),
        in_specs=[
            pl.BlockSpec((_NC, _BN, _D), lambda i: (0, i, 0)),
            pl.BlockSpec((_NW, _RCNT, 16), lambda i: (0, 0, 0)),
            pl.BlockSpec((_BN, _D), lambda i: (i, 0)),
            pl.BlockSpec((_D, _D), lambda i: (0, 0)),
            pl.BlockSpec((1, _D), lambda i: (0, 0)),
            pl.BlockSpec((_D, _D), lambda i: (0, 0)),
        ],
        out_specs=[pl.BlockSpec((_BN, _D), lambda i: (i, 0)),
                   pl.BlockSpec((_BN, 8), lambda i: (i, 0))],
        out_shape=[jax.ShapeDtypeStruct((_N, _D), jnp.float32),
                   jax.ShapeDtypeStruct((_N, 8), jnp.float32)],
    )(aggp, cparts, feats, wl, b.reshape(1, _D), wr)


def _dense2_body(agg_ref, cnt_ref, x_ref, wl_ref, b_ref, wr_ref, o_ref):
    o_ref[...] = _sage_out(agg_ref, cnt_ref[:, 0:1], x_ref, wl_ref, b_ref,
                           wr_ref, False)


def _dense2(aggp, cnt8, feats, wl, b, wr):
    return pl.pallas_call(
        _dense2_body,
        grid=(_N // _BN,),
        in_specs=[
            pl.BlockSpec((_NC, _BN, _D), lambda i: (0, i, 0)),
            pl.BlockSpec((_BN, 8), lambda i: (i, 0)),
            pl.BlockSpec((_BN, _D), lambda i: (i, 0)),
            pl.BlockSpec((_D, _D), lambda i: (0, 0)),
            pl.BlockSpec((1, _D), lambda i: (0, 0)),
            pl.BlockSpec((_D, _D), lambda i: (0, 0)),
        ],
        out_specs=pl.BlockSpec((_BN, _D), lambda i: (i, 0)),
        out_shape=jax.ShapeDtypeStruct((_N, _D), jnp.float32),
    )(aggp, cnt8, feats, wl, b.reshape(1, _D), wr)


def kernel(x, edge_index, W1_l, b1_l, W1_r, W2_l, b2_l, W2_r):
    src = edge_index[0]
    dst = edge_index[1]
    zrow = jnp.zeros((_ZB, _D), jnp.float32)

    (cparts,) = _cnt_parts(dst)
    (agg1,) = _agg_plain(x, src, dst, zrow)
    h, cnt8 = _dense1(agg1, cparts, x, W1_l, b1_l, W1_r)
    (agg2,) = _agg_plain(h, src, dst, zrow)
    out = _dense2(agg2, cnt8, h, W2_l, b2_l, W2_r)
    return out


# speedup vs baseline: 1.0382x; 1.0382x over previous
import functools

import jax
import jax.numpy as jnp
from jax import lax
from jax.experimental import pallas as pl
from jax.experimental.pallas import tpu as pltpu
from jax.experimental.pallas import tpu_sc as plsc

_N = 10000
_E = 320000
_D = 128
_NC = 2
_NS = 16
_NW = _NC * _NS
_EPW = _E // _NW
_C = 128
_G = _EPW // _C
_CT = _EPW - _G * _C
_RPT = 624
_TAIL = _N - _RPT * _NS
_ZB = 48

assert _EPW * _NW == _E and 0 <= _TAIL < _RPT
assert _G % 2 == 0 and _G >= 4 and 0 < _CT and _CT % 8 == 0
assert _RPT % 8 == 0 and _TAIL % 8 == 0
assert _RPT % _ZB == 0 and _TAIL <= _ZB
_ZG = _RPT // _ZB
_RCNT = _N // 16
assert _N % 16 == 0 and _CT == 16


def _make_agg():
    out_type = [jax.ShapeDtypeStruct((_NC, _N, _D), jnp.float32)]
    scratch = [
        pltpu.VMEM((2, _C), jnp.int32),
        pltpu.VMEM((2, _C), jnp.int32),
        pltpu.VMEM((_C, _D), jnp.float32),
        pltpu.VMEM((_C, _D), jnp.float32),
        pltpu.VMEM((_CT,), jnp.int32),
        pltpu.VMEM((_CT,), jnp.int32),
        pltpu.VMEM((_CT, _D), jnp.float32),
        pltpu.VMEM((_ZB, _D), jnp.float32),
        pltpu.VMEM_SHARED((_N, _D), jnp.float32),
        pltpu.SemaphoreType.DMA,
        pltpu.SemaphoreType.DMA,
        pltpu.SemaphoreType.DMA,
        pltpu.SemaphoreType.DMA,
    ]

    mesh = plsc.VectorSubcoreMesh(core_axis_name="c", subcore_axis_name="s")

    def body(*refs):
        (feats, srch, dsth, zrow,
         aggout,
         srcv, dstv, rows0, rows1, srct, dstt, rowst,
         bounce, aggsh, sem0, sem1, ssem0, ssem1) = refs
        rows = (rows0, rows1)
        sems = (sem0, sem1)
        ssems = (ssem0, ssem1)
        cid = lax.axis_index("c")
        sid = lax.axis_index("s")
        wid = cid * _NS + sid
        r0 = sid * _RPT
        ebase = wid * _EPW

        def zero_slice():
            pltpu.sync_copy(zrow, bounce)
            for j in range(_ZG):
                pltpu.sync_copy(bounce, aggsh.at[pl.ds(r0 + j * _ZB, _ZB)])

            @pl.when(sid == _NS - 1)
            def _():
                pltpu.sync_copy(bounce.at[pl.ds(0, _TAIL)],
                                aggsh.at[pl.ds(_RPT * _NS, _TAIL)])

        def export_slice(out):
            for j in range(_ZG):
                pltpu.sync_copy(aggsh.at[pl.ds(r0 + j * _ZB, _ZB)], bounce)
                pltpu.sync_copy(bounce, out.at[cid, pl.ds(r0 + j * _ZB, _ZB)])

            @pl.when(sid == _NS - 1)
            def _():
                pltpu.sync_copy(aggsh.at[pl.ds(_RPT * _NS, _TAIL)],
                                bounce.at[pl.ds(0, _TAIL)])
                pltpu.sync_copy(bounce.at[pl.ds(0, _TAIL)],
                                out.at[cid, pl.ds(_RPT * _NS, _TAIL)])

        def load_idx(off, slot):
            pltpu.sync_copy(srch.at[pl.ds(off, _C)], srcv.at[slot])
            pltpu.sync_copy(dsth.at[pl.ds(off, _C)], dstv.at[slot])

        def gather(slot):
            return pltpu.make_async_copy(feats.at[srcv.at[slot]],
                                         rows[slot], sems[slot])

        def scat(slot):
            return pltpu.make_async_copy(rows[slot], aggsh.at[dstv.at[slot]],
                                         ssems[slot])

        zero_slice()
        plsc.subcore_barrier()

        load_idx(ebase, 0)
        gather(0).start()
        load_idx(ebase + _C, 1)
        gather(1).start()

        def step(g, carry):
            ca = 2 * g
            gather(0).wait()
            scat(0).start(add=True)
            gather(1).wait()
            scat(1).start(add=True)
            scat(0).wait()
            load_idx(ebase + (ca + 2) * _C, 0)
            gather(0).start()
            scat(1).wait()
            load_idx(ebase + (ca + 3) * _C, 1)
            gather(1).start()
            return carry

        lax.fori_loop(0, (_G - 2) // 2, step, 0)
        gather(0).wait()
        scat(0).start(add=True)
        gather(1).wait()
        scat(1).start(add=True)
        tbase = ebase + _G * _C
        pltpu.sync_copy(srch.at[pl.ds(tbase, _CT)], srct)
        pltpu.sync_copy(dsth.at[pl.ds(tbase, _CT)], dstt)
        pltpu.make_async_copy(feats.at[srct], rowst, sem0).start()
        scat(0).wait()
        scat(1).wait()
        pltpu.make_async_copy(feats.at[srct], rowst, sem0).wait()
        pltpu.sync_copy(rowst, aggsh.at[dstt], add=True)
        plsc.subcore_barrier()
        export_slice(aggout)

    return pl.kernel(body, mesh=mesh, out_type=out_type,
                     scratch_types=scratch)


def _make_cnt():
    CC = 2000

    def body(dsth, cntout, dstv, cnt2d, sem):
        cid = lax.axis_index("c")
        sid = lax.axis_index("s")
        wid = cid * _NS + sid
        ebase = wid * _EPW
        z16 = jnp.zeros((16,), jnp.float32)
        ones16 = jnp.ones((16,), jnp.float32)

        def zc(i, carry):
            cnt2d[i, pl.ds(0, 16)] = z16
            return carry

        lax.fori_loop(0, _RCNT, zc, 0)

        def kstep(k, carry):
            pltpu.sync_copy(dsth.at[pl.ds(ebase + k * CC, CC)], dstv)
            for i in range(CC // 16):
                dvec = dstv[pl.ds(i * 16, 16)]
                ri = lax.shift_right_logical(dvec, 4)
                li = lax.bitwise_and(dvec, 15)
                plsc.addupdate_scatter(cnt2d, [ri, li], ones16)
            return carry

        lax.fori_loop(0, _EPW // CC, kstep, 0)
        pltpu.sync_copy(cnt2d, cntout.at[wid])

    mesh = plsc.VectorSubcoreMesh(core_axis_name="c", subcore_axis_name="s")
    return pl.kernel(
        body, mesh=mesh,
        out_type=[jax.ShapeDtypeStruct((_NW, _RCNT, 16), jnp.float32)],
        compiler_params=pltpu.CompilerParams(needs_layout_passes=False),
        scratch_types=[
            pltpu.VMEM((CC,), jnp.int32),
            pltpu.VMEM((_RCNT, 16), jnp.float32),
            pltpu.SemaphoreType.DMA,
        ])


_agg_plain = _make_agg()
_cnt_parts = _make_cnt()

_BN = 1000


def _merged_counts(c_ref, i):
    s = jnp.sum(c_ref[...], axis=0)
    n_row = jax.lax.broadcasted_iota(jnp.int32, (_BN, _RCNT), 0) + i * _BN
    r_col = jax.lax.broadcasted_iota(jnp.int32, (_BN, _RCNT), 1)
    pick_row = (lax.shift_right_logical(n_row, 4) == r_col)
    t = lax.dot_general(pick_row.astype(jnp.float32), s,
                        (((1,), (0,)), ((), ())),
                        preferred_element_type=jnp.float32)
    n_lane = jax.lax.broadcasted_iota(jnp.int32, (_BN, 16), 0) + i * _BN
    l_lane = jax.lax.broadcasted_iota(jnp.int32, (_BN, 16), 1)
    sel = jnp.where(lax.bitwise_and(n_lane, 15) == l_lane, t, 0.0)
    return jnp.sum(sel, axis=1, keepdims=True)


def _sage_out(agg_ref, cnt, x_ref, wl_ref, b_ref, wr_ref, relu):
    agg = agg_ref[0] + agg_ref[1]
    mean = agg / jnp.maximum(cnt, 1.0)
    dn = (((1,), (0,)), ((), ()))
    acc = (lax.dot_general(mean, wl_ref[...], dn,
                           preferred_element_type=jnp.float32,
                           precision=lax.Precision.HIGHEST)
           + b_ref[...]
           + lax.dot_general(x_ref[...], wr_ref[...], dn,
                             preferred_element_type=jnp.float32,
                             precision=lax.Precision.HIGHEST))
    return jnp.maximum(acc, 0.0) if relu else acc


def _cnt_merge_body(c_ref, o_ref):
    o_ref[...] = jnp.broadcast_to(_merged_counts(c_ref, pl.program_id(0)),
                                  (_BN, 8))


def _cnt_merge(cparts):
    return pl.pallas_call(
        _cnt_merge_body,
        grid=(_N // _BN,),
        in_specs=[pl.BlockSpec((_NW, _RCNT, 16), lambda i: (0, 0, 0))],
        out_specs=pl.BlockSpec((_BN, 8), lambda i: (i, 0)),
        out_shape=jax.ShapeDtypeStruct((_N, 8), jnp.float32),
    )(cparts)


def _dense_body(relu, agg_ref, cnt_ref, x_ref, wl_ref, b_ref, wr_ref, o_ref):
    o_ref[...] = _sage_out(agg_ref, cnt_ref[:, 0:1], x_ref, wl_ref, b_ref,
                           wr_ref, relu)


def _dense(aggp, cnt8, feats, wl, b, wr, relu):
    return pl.pallas_call(
        functools.partial(_dense_body, relu),
        grid=(_N // _BN,),
        in_specs=[
            pl.BlockSpec((_NC, _BN, _D), lambda i: (0, i, 0)),
            pl.BlockSpec((_BN, 8), lambda i: (i, 0)),
            pl.BlockSpec((_BN, _D), lambda i: (i, 0)),
            pl.BlockSpec((_D, _D), lambda i: (0, 0)),
            pl.BlockSpec((1, _D), lambda i: (0, 0)),
            pl.BlockSpec((_D, _D), lambda i: (0, 0)),
        ],
        out_specs=pl.BlockSpec((_BN, _D), lambda i: (i, 0)),
        out_shape=jax.ShapeDtypeStruct((_N, _D), jnp.float32),
    )(aggp, cnt8, feats, wl, b.reshape(1, _D), wr)


def kernel(x, edge_index, W1_l, b1_l, W1_r, W2_l, b2_l, W2_r):
    src = edge_index[0]
    dst = edge_index[1]
    zrow = jnp.zeros((_ZB, _D), jnp.float32)

    (cparts,) = _cnt_parts(dst)
    (agg1,) = _agg_plain(x, src, dst, zrow)
    cnt8 = _cnt_merge(cparts)
    h = _dense(agg1, cnt8, x, W1_l, b1_l, W1_r, relu=True)
    (agg2,) = _agg_plain(h, src, dst, zrow)
    out = _dense(agg2, cnt8, h, W2_l, b2_l, W2_r, relu=False)
    return out
